# baseline (speedup 1.0000x reference)
import jax
import jax.numpy as jnp
from jax.experimental import pallas as pl


def _matmul_body(x_ref, w_ref, o_ref):
    o_ref[...] = jnp.dot(x_ref[...], w_ref[...],
                         preferred_element_type=jnp.float32)


def kernel(x, edge_index, adj_values, features_nonzero, W):
    n, d_in = x.shape
    d_out = W.shape[1]
    blk = 1000
    h = pl.pallas_call(
        _matmul_body,
        grid=(n // blk,),
        in_specs=[
            pl.BlockSpec((blk, d_in), lambda i: (i, 0)),
            pl.BlockSpec((d_in, d_out), lambda i: (0, 0)),
        ],
        out_specs=pl.BlockSpec((blk, d_out), lambda i: (i, 0)),
        out_shape=jax.ShapeDtypeStruct((n, d_out), jnp.float32),
    )(x, W)
    row = edge_index[0].astype(jnp.int32)
    col = edge_index[1].astype(jnp.int32)
    msg = h[col] * adj_values[:, None]
    out = jax.ops.segment_sum(msg, row, num_segments=n)
    return jax.nn.relu(out)


# speedup vs baseline: 3.5861x; 3.5861x over previous
import dataclasses
import functools

import jax
import jax.numpy as jnp
from jax import lax
from jax.experimental import pallas as pl
from jax.experimental.pallas import tpu as pltpu
from jax.experimental.pallas import tpu_sc as plsc

N = 10000
NPAD = 10240
E = 160000
D = 256
DH = 128
NS = 16
NC = 2
PT = E // NS
CH = 80
NCHUNK = PT // CH
NBLK = 5
BCH = NCHUNK // NBLK
RPT = NPAD // NS
WCH = 80


def _matmul_body(x_ref, w_ref, o_ref):
    o_ref[...] = jnp.dot(x_ref[...], w_ref[...],
                         preferred_element_type=jnp.float32)


def _compute_hsplit(x, W):
    return pl.pallas_call(
        _matmul_body,
        grid=(10, 2),
        in_specs=[
            pl.BlockSpec((1000, D), lambda i, j: (i, 0)),
            pl.BlockSpec((D, DH), lambda i, j: (0, j)),
        ],
        out_specs=pl.BlockSpec((1000, DH), lambda i, j: (j * 10 + i, 0)),
        out_shape=jax.ShapeDtypeStruct((NC * N, DH), jnp.float32),
    )(x, W)


_vector_mesh = plsc.VectorSubcoreMesh(core_axis_name="c", subcore_axis_name="s")

_sc_compiler_params = pltpu.CompilerParams()
if "needs_layout_passes" in pltpu.CompilerParams.__dataclass_fields__:
    _sc_compiler_params = dataclasses.replace(
        _sc_compiler_params, needs_layout_passes=False)


@functools.partial(
    pl.kernel,
    out_type=jax.ShapeDtypeStruct((N, D), jnp.float32),
    mesh=_vector_mesh,
    compiler_params=_sc_compiler_params,
    scratch_types=[
        pltpu.VMEM_SHARED((NPAD, DH), jnp.float32),
        pltpu.VMEM((BCH, CH), jnp.int32),
        pltpu.VMEM((BCH, CH), jnp.int32),
        pltpu.VMEM((BCH, CH), jnp.float32),
        pltpu.VMEM((CH, DH), jnp.float32),
        pltpu.VMEM((WCH, DH), jnp.float32),
        pltpu.SemaphoreType.DMA,
    ],
)
def _sc_aggregate(h_hbm, row_hbm, col_hbm, adj_hbm, z_hbm, out_hbm,
                  accum, row_scr, col_scr, val_scr, gbuf, stage, sem):
    c = lax.axis_index("c")
    s = lax.axis_index("s")
    r0 = s * RPT

    pltpu.sync_copy(z_hbm.at[pl.ds(r0, RPT)], accum.at[pl.ds(r0, RPT)])
    plsc.subcore_barrier()

    @pl.loop(0, NBLK)
    def _block(b):
        pltpu.sync_copy(row_hbm.at[s, b], row_scr)
        pltpu.sync_copy(col_hbm.at[c, s, b], col_scr)
        pltpu.sync_copy(adj_hbm.at[s, b], val_scr)

        @pl.loop(0, BCH)
        def _chunk(j):
            pltpu.async_copy(h_hbm.at[col_scr.at[j]], gbuf, sem).wait()

            @pl.loop(0, CH)
            def _row(r):
                v = plsc.load_gather(
                    val_scr,
                    [jnp.full((16,), j, jnp.int32),
                     jnp.full((16,), r, jnp.int32)])
                for k in range(DH // 16):
                    sl = (r, pl.ds(k * 16, 16))
                    gbuf[sl] = gbuf[sl] * v

            pltpu.sync_copy(gbuf, accum.at[row_scr.at[j]], add=True)

    plsc.subcore_barrier()

    for t in range(RPT // WCH):
        base = r0 + t * WCH

        @pl.when(base < N)
        def _write():
            pltpu.sync_copy(accum.at[pl.ds(base, WCH)], stage)

            @pl.loop(0, WCH)
            def _relu_row(r):
                for k in range(DH // 16):
                    sl = (r, pl.ds(k * 16, 16))
                    stage[sl] = jnp.maximum(stage[sl], 0.0)

            pltpu.sync_copy(
                stage, out_hbm.at[pl.ds(base, WCH), pl.ds(c * DH, DH)])


def kernel(x, edge_index, adj_values, features_nonzero, W):
    row = edge_index[0].astype(jnp.int32)
    col = edge_index[1].astype(jnp.int32)
    hsplit = _compute_hsplit(x, W)
    row4 = row.reshape(NS, NBLK, BCH, CH)
    col5 = jnp.stack([col, col + N]).reshape(NC, NS, NBLK, BCH, CH)
    adj4 = adj_values.reshape(NS, NBLK, BCH, CH)
    zeros = jnp.zeros((NPAD, DH), jnp.float32)
    return _sc_aggregate(hsplit, row4, col5, adj4, zeros)
